# baseline (speedup 1.0000x reference)
import functools

import jax
import jax.numpy as jnp
from jax import lax
from jax.experimental import pallas as pl
from jax.experimental.pallas import tpu as pltpu
from jax.experimental.pallas import tpu_sc as plsc

B = 16384
ED = 32
CD = 64
NU = 1000000
NI = 100000
PW = 128

OWN = 31232
STREAM = 32768
CHUNK = 1024
NCHUNK = STREAM // CHUNK
NG = 3
MAIN_END = 999936
TAIL_LO = NU - 128
NIDV = B // 16
CAP = 2048
CCAP = 256
DUMP = B


def _pack_body(cmT_ref, itT_ref, wT_ref, b_ref, p_ref):
    cm_blk = cmT_ref[...]
    proj = jax.lax.dot_general(
        cm_blk, wT_ref[...],
        dimension_numbers=(((0,), (1,)), ((), ())),
        preferred_element_type=jnp.float32,
    )
    proj = jnp.maximum(proj + b_ref[...][None, :], 0.0)
    item = itT_ref[...].T
    blk = item.shape[0]
    pad = jnp.zeros((blk, PW - 2 * ED), jnp.float32)
    p_ref[...] = jnp.concatenate([item, proj, pad], axis=1)


def _pack_p(cmT, itT, wT, b):
    blk = 12800
    grid = (NI + blk - 1) // blk
    return pl.pallas_call(
        _pack_body,
        grid=(grid,),
        in_specs=[
            pl.BlockSpec((CD, blk), lambda i: (0, i)),
            pl.BlockSpec((ED, blk), lambda i: (0, i)),
            pl.BlockSpec((ED, CD), lambda i: (0, 0)),
            pl.BlockSpec((ED,), lambda i: (0,)),
        ],
        out_specs=pl.BlockSpec((blk, PW), lambda i: (i, 0)),
        out_shape=jax.ShapeDtypeStruct((NI, PW), jnp.float32),
    )(cmT, itT, wT, b)


_info = plsc.get_sparse_core_info()
_NW = _info.num_cores * _info.num_subcores
_BPW = B // _NW


def _compact(src_ref, n_vregs, predicate, out_refs, make_vals, cap,
             unroll=None):
    def body(j, carry):
        c = src_ref[pl.ds(j * 16, 16)]
        m = predicate(c)
        cs = plsc.cumsum(jnp.where(m, 1, 0).astype(jnp.int32))
        pos = carry + cs - 1
        mm = m & (pos < cap)
        for out_ref, vals in zip(out_refs, make_vals(j, c)):
            plsc.store_scatter(out_ref, [pos], vals, mask=mm)
        return carry + lax.reduce_max(cs, (0,))

    if unroll is not None:
        return lax.fori_loop(0, n_vregs, body, jnp.int32(0), unroll=unroll)
    return lax.fori_loop(0, n_vregs, body, jnp.int32(0))


def _make_sc_user_gather():
    mesh = plsc.VectorSubcoreMesh(core_axis_name="c", subcore_axis_name="s")
    iota16 = lambda: lax.iota(jnp.int32, 16)

    @functools.partial(
        pl.kernel,
        mesh=mesh,
        compiler_params=pltpu.CompilerParams(
            use_tc_tiling_on_sc=True, needs_layout_passes=False),
        out_type=jax.ShapeDtypeStruct((B + 16, PW), jnp.float32),
        scratch_types=[
            pltpu.VMEM((B,), jnp.int32),
            pltpu.VMEM((CAP + 16,), jnp.int32),
            pltpu.VMEM((CAP + 16,), jnp.int32),
            pltpu.VMEM((CCAP + 16,), jnp.int32),
            pltpu.VMEM((CCAP + 16,), jnp.int32),
            pltpu.VMEM((ED, CHUNK), jnp.float32),
            pltpu.VMEM((ED, CHUNK), jnp.float32),
            pltpu.VMEM((ED, 128), jnp.float32),
            pltpu.VMEM((NG * 16, PW), jnp.float32),
            pltpu.VMEM((NG * 16, PW), jnp.float32),
            pltpu.VMEM((8, 16), jnp.int32),
            pltpu.VMEM((8, 16), jnp.int32),
            pltpu.SemaphoreType.DMA,
            pltpu.SemaphoreType.DMA,
            pltpu.SemaphoreType.DMA,
        ],
    )
    def user_gather(ids, tableT, tailT, out, ids_v, cbuf, bbuf, ccb, bcb,
                    slab_a, slab_b, tail_v, rows_0, rows_1, idxr_0, idxr_1,
                    s_a, s_b, s_out):
        wid = lax.axis_index("s") * _info.num_cores + lax.axis_index("c")
        lo = wid * OWN
        hi = jnp.where(wid == _NW - 1, NU, lo + OWN)
        off = jnp.minimum(lo, MAIN_END - STREAM)

        def issue(slab, sem, c_lo):
            return pltpu.async_copy(
                tableT.at[:, pl.ds(pl.multiple_of(c_lo, 128), CHUNK)],
                slab, sem)

        pltpu.sync_copy(ids, ids_v)
        pltpu.sync_copy(tailT, tail_v)

        n = _compact(
            ids_v, NIDV, lambda c: (c >= lo) & (c < hi),
            [cbuf, bbuf],
            lambda j, c: [c, j * 16 + iota16()],
            CAP,
        )
        n = jnp.minimum(n, CAP)
        nv = (n + 15) // 16
        sent = n + lax.iota(jnp.int32, 16)
        plsc.store_scatter(cbuf, [sent], jnp.full((16,), -1, jnp.int32))
        plsc.store_scatter(bbuf, [sent], jnp.full((16,), DUMP, jnp.int32))

        rows_p = [rows_0, rows_1]
        idxr_p = [idxr_0, idxr_1]

        def process(slab, c_lo, width, par, drain):
            rows_v, idxr = rows_p[par], idxr_p[par]

            if drain is True:
                for g in range(NG):
                    pltpu.make_async_copy(
                        rows_v.at[pl.ds(g * 16, 16)],
                        out.at[idxr.at[g]], s_out).wait()
            elif drain is not False:
                @pl.when(drain)
                def _():
                    for g in range(NG):
                        pltpu.make_async_copy(
                            rows_v.at[pl.ds(g * 16, 16)],
                            out.at[idxr.at[g]], s_out).wait()

            m = _compact(
                cbuf, nv,
                lambda c: (c >= c_lo) & (c < c_lo + width),
                [ccb, bcb],
                lambda j, c: [c - c_lo,
                              plsc.load_gather(bbuf, [j * 16 + iota16()])],
                CCAP,
            )
            m = jnp.minimum(m, NG * 16)

            for g in range(NG):
                lmask = g * 16 + iota16() < m
                cc = plsc.load_gather(ccb, [g * 16 + iota16()])
                bb = plsc.load_gather(bcb, [g * 16 + iota16()])
                cc = jnp.where(lmask, cc, 0)
                bb = jnp.where(lmask, bb, DUMP)
                for f in range(ED):
                    vals = plsc.load_gather(
                        slab, [jnp.full((16,), f, jnp.int32), cc])
                    plsc.store_scatter(
                        rows_v.at[pl.ds(g * 16, 16)],
                        [iota16(), jnp.full((16,), f, jnp.int32)], vals)
                idxr[g, :] = bb
                pltpu.async_copy(
                    rows_v.at[pl.ds(g * 16, 16)], out.at[idxr.at[g]],
                    s_out).wait()

        def chunk_body(ti, _):
            c_lo = off + ti * CHUNK
            pltpu.sync_copy(
                tableT.at[:, pl.ds(pl.multiple_of(c_lo, 128), CHUNK)],
                slab_a)
            process(slab_a, c_lo, CHUNK, 0, False)
            return 0

        lax.fori_loop(0, NCHUNK, chunk_body, 0)

        process(tail_v, TAIL_LO, 128, NCHUNK % 2, False)


    return user_gather


def _make_sc_p_gather():
    mesh = plsc.VectorSubcoreMesh(core_axis_name="c", subcore_axis_name="s")

    @functools.partial(
        pl.kernel,
        mesh=mesh,
        compiler_params=pltpu.CompilerParams(use_tc_tiling_on_sc=True),
        out_type=jax.ShapeDtypeStruct((B, PW), jnp.float32),
        scratch_types=[
            pltpu.VMEM((_BPW,), jnp.int32),
            pltpu.VMEM((_BPW, PW), jnp.float32),
            pltpu.SemaphoreType.DMA,
        ],
    )
    def p_gather(ids, ptable, out, idx_v, rows_v, sem):
        wid = lax.axis_index("s") * _info.num_cores + lax.axis_index("c")
        base = wid * _BPW
        pltpu.sync_copy(ids.at[pl.ds(base, _BPW)], idx_v)
        pltpu.async_copy(ptable.at[idx_v], rows_v, sem).wait()
        pltpu.sync_copy(rows_v, out.at[pl.ds(base, _BPW)])

    return p_gather


_sc_user_gather = _make_sc_user_gather()
_sc_p_gather = _make_sc_p_gather()


def _score_body(u_ref, pg_ref, out_ref):
    u = u_ref[:, :ED]
    s = pg_ref[:, :ED] + pg_ref[:, ED:2 * ED]
    out_ref[...] = jnp.sum(u * s, axis=1)


def _score(u_rows, pg):
    blk = 2048
    grid = B // blk
    return pl.pallas_call(
        _score_body,
        grid=(grid,),
        in_specs=[
            pl.BlockSpec((blk, PW), lambda i: (i, 0)),
            pl.BlockSpec((blk, PW), lambda i: (i, 0)),
        ],
        out_specs=pl.BlockSpec((blk,), lambda i: (i,)),
        out_shape=jax.ShapeDtypeStruct((B,), jnp.float32),
    )(u_rows, pg)


def kernel(user_ids, item_ids, user_table, item_table, content_matrix, W, b):
    user_ids = user_ids.astype(jnp.int32)
    item_ids = item_ids.astype(jnp.int32)
    p = _pack_p(content_matrix.T, item_table.T, W.T, b)
    u_rows = _sc_user_gather(user_ids, user_table.T,
                             user_table.T[:, TAIL_LO:])
    pg = _sc_p_gather(item_ids, p)
    return _score(u_rows, pg)


# speedup vs baseline: 3.0175x; 3.0175x over previous
import functools

import jax
import jax.numpy as jnp
from jax import lax
from jax.experimental import pallas as pl
from jax.experimental.pallas import tpu as pltpu
from jax.experimental.pallas import tpu_sc as plsc

B = 16384
ED = 32
CD = 64
NU = 1000000
NI = 100000
PW = 128

OWN = 31232
STREAM = 32768
CHUNK = 1024
NCHUNK = STREAM // CHUNK
NG = 3
MAIN_END = 999936
TAIL_LO = NU - 128
NIDV = B // 16
CAP = 2048
CCAP = 256
DUMP = B


def _pack_body(cmT_ref, itT_ref, wT_ref, b_ref, p_ref):
    cm_blk = cmT_ref[...]
    proj = jax.lax.dot_general(
        cm_blk, wT_ref[...],
        dimension_numbers=(((0,), (1,)), ((), ())),
        preferred_element_type=jnp.float32,
    )
    proj = jnp.maximum(proj + b_ref[...][None, :], 0.0)
    item = itT_ref[...].T
    blk = item.shape[0]
    pad = jnp.zeros((blk, PW - 2 * ED), jnp.float32)
    p_ref[...] = jnp.concatenate([item, proj, pad], axis=1)


def _pack_p(cmT, itT, wT, b):
    blk = 12800
    grid = (NI + blk - 1) // blk
    return pl.pallas_call(
        _pack_body,
        grid=(grid,),
        in_specs=[
            pl.BlockSpec((CD, blk), lambda i: (0, i)),
            pl.BlockSpec((ED, blk), lambda i: (0, i)),
            pl.BlockSpec((ED, CD), lambda i: (0, 0)),
            pl.BlockSpec((ED,), lambda i: (0,)),
        ],
        out_specs=pl.BlockSpec((blk, PW), lambda i: (i, 0)),
        out_shape=jax.ShapeDtypeStruct((NI, PW), jnp.float32),
    )(cmT, itT, wT, b)


_info = plsc.get_sparse_core_info()
_NW = _info.num_cores * _info.num_subcores
_BPW = B // _NW


def _compact(src_ref, n_vregs, predicate, out_refs, make_vals, cap,
             unroll=None):
    def body(j, carry):
        c = src_ref[pl.ds(j * 16, 16)]
        m = predicate(c)
        cs = plsc.cumsum(jnp.where(m, 1, 0).astype(jnp.int32))
        pos = carry + cs - 1
        mm = m & (pos < cap)
        for out_ref, vals in zip(out_refs, make_vals(j, c)):
            plsc.store_scatter(out_ref, [pos], vals, mask=mm)
        return carry + lax.reduce_max(cs, (0,))

    if unroll is not None:
        return lax.fori_loop(0, n_vregs, body, jnp.int32(0), unroll=unroll)
    return lax.fori_loop(0, n_vregs, body, jnp.int32(0))


def _make_sc_user_gather():
    mesh = plsc.VectorSubcoreMesh(core_axis_name="c", subcore_axis_name="s")
    iota16 = lambda: lax.iota(jnp.int32, 16)

    @functools.partial(
        pl.kernel,
        mesh=mesh,
        compiler_params=pltpu.CompilerParams(
            use_tc_tiling_on_sc=True, needs_layout_passes=False),
        out_type=jax.ShapeDtypeStruct((B + 16, PW), jnp.float32),
        scratch_types=[
            pltpu.VMEM((B,), jnp.int32),
            pltpu.VMEM((CAP + 16,), jnp.int32),
            pltpu.VMEM((CAP + 16,), jnp.int32),
            pltpu.VMEM((CCAP + 16,), jnp.int32),
            pltpu.VMEM((CCAP + 16,), jnp.int32),
            pltpu.VMEM((ED, CHUNK), jnp.float32),
            pltpu.VMEM((ED, CHUNK), jnp.float32),
            pltpu.VMEM((ED, 128), jnp.float32),
            pltpu.VMEM((16, PW), jnp.float32),
            pltpu.VMEM((16, PW), jnp.float32),
            pltpu.VMEM((8, 16), jnp.int32),
            pltpu.VMEM((8, 16), jnp.int32),
            pltpu.SemaphoreType.DMA,
            pltpu.SemaphoreType.DMA,
            pltpu.SemaphoreType.DMA,
        ],
    )
    def user_gather(ids, tableT, tailT, out, ids_v, cbuf, bbuf, ccb, bcb,
                    slab_a, slab_b, tail_v, rows_0, rows_1, idxr_0, idxr_1,
                    s_a, s_b, s_out):
        wid = lax.axis_index("s") * _info.num_cores + lax.axis_index("c")
        lo = wid * OWN
        hi = jnp.where(wid == _NW - 1, NU, lo + OWN)
        off = jnp.minimum(lo, MAIN_END - STREAM)

        def issue(slab, sem, c_lo):
            return pltpu.async_copy(
                tableT.at[:, pl.ds(pl.multiple_of(c_lo, 128), CHUNK)],
                slab, sem)

        pltpu.sync_copy(ids, ids_v)
        pltpu.sync_copy(tailT, tail_v)

        n = _compact(
            ids_v, NIDV, lambda c: (c >= lo) & (c < hi),
            [cbuf, bbuf],
            lambda j, c: [c, j * 16 + iota16()],
            CAP,
        )
        n = jnp.minimum(n, CAP)
        nv = (n + 15) // 16
        sent = n + lax.iota(jnp.int32, 16)
        plsc.store_scatter(cbuf, [sent], jnp.full((16,), -1, jnp.int32))
        plsc.store_scatter(bbuf, [sent], jnp.full((16,), DUMP, jnp.int32))

        rows_p = [rows_0, rows_1]
        idxr_p = [idxr_0, idxr_1]

        def process(slab, c_lo, width, par, drain):
            rows_v, idxr = rows_p[par], idxr_p[par]

            if drain is True:
                for g in range(NG):
                    pltpu.make_async_copy(
                        rows_v.at[pl.ds(g * 16, 16)],
                        out.at[idxr.at[g]], s_out).wait()
            elif drain is not False:
                @pl.when(drain)
                def _():
                    for g in range(NG):
                        pltpu.make_async_copy(
                            rows_v.at[pl.ds(g * 16, 16)],
                            out.at[idxr.at[g]], s_out).wait()

            m = _compact(
                cbuf, nv,
                lambda c: (c >= c_lo) & (c < c_lo + width),
                [ccb, bcb],
                lambda j, c: [c - c_lo,
                              plsc.load_gather(bbuf, [j * 16 + iota16()])],
                CCAP,
            )
            m = jnp.minimum(m, NG * 16)

            def group(g, _):
                lmask = g * 16 + iota16() < m
                cc = plsc.load_gather(ccb, [g * 16 + iota16()])
                bb = plsc.load_gather(bcb, [g * 16 + iota16()])
                cc = jnp.where(lmask, cc, 0)
                bb = jnp.where(lmask, bb, DUMP)
                for f in range(ED):
                    vals = plsc.load_gather(
                        slab, [jnp.full((16,), f, jnp.int32), cc])
                    plsc.store_scatter(
                        rows_v, [iota16(), jnp.full((16,), f, jnp.int32)],
                        vals)
                idxr[0, :] = bb
                pltpu.async_copy(
                    rows_v, out.at[idxr.at[0]], s_out).wait()
                return 0

            lax.fori_loop(0, (m + 15) // 16, group, 0)

        def chunk_body(ti, _):
            c_lo = off + ti * CHUNK
            pltpu.sync_copy(
                tableT.at[:, pl.ds(pl.multiple_of(c_lo, 128), CHUNK)],
                slab_a)
            process(slab_a, c_lo, CHUNK, 0, False)
            return 0

        lax.fori_loop(0, NCHUNK, chunk_body, 0)

        process(tail_v, TAIL_LO, 128, NCHUNK % 2, False)


    return user_gather


def _make_sc_p_gather():
    mesh = plsc.VectorSubcoreMesh(core_axis_name="c", subcore_axis_name="s")

    @functools.partial(
        pl.kernel,
        mesh=mesh,
        compiler_params=pltpu.CompilerParams(use_tc_tiling_on_sc=True),
        out_type=jax.ShapeDtypeStruct((B, PW), jnp.float32),
        scratch_types=[
            pltpu.VMEM((_BPW,), jnp.int32),
            pltpu.VMEM((_BPW, PW), jnp.float32),
            pltpu.SemaphoreType.DMA,
        ],
    )
    def p_gather(ids, ptable, out, idx_v, rows_v, sem):
        wid = lax.axis_index("s") * _info.num_cores + lax.axis_index("c")
        base = wid * _BPW
        pltpu.sync_copy(ids.at[pl.ds(base, _BPW)], idx_v)
        pltpu.async_copy(ptable.at[idx_v], rows_v, sem).wait()
        pltpu.sync_copy(rows_v, out.at[pl.ds(base, _BPW)])

    return p_gather


_sc_user_gather = _make_sc_user_gather()
_sc_p_gather = _make_sc_p_gather()


def _score_body(u_ref, pg_ref, out_ref):
    u = u_ref[:, :ED]
    s = pg_ref[:, :ED] + pg_ref[:, ED:2 * ED]
    out_ref[...] = jnp.sum(u * s, axis=1)


def _score(u_rows, pg):
    blk = 2048
    grid = B // blk
    return pl.pallas_call(
        _score_body,
        grid=(grid,),
        in_specs=[
            pl.BlockSpec((blk, PW), lambda i: (i, 0)),
            pl.BlockSpec((blk, PW), lambda i: (i, 0)),
        ],
        out_specs=pl.BlockSpec((blk,), lambda i: (i,)),
        out_shape=jax.ShapeDtypeStruct((B,), jnp.float32),
    )(u_rows, pg)


def kernel(user_ids, item_ids, user_table, item_table, content_matrix, W, b):
    user_ids = user_ids.astype(jnp.int32)
    item_ids = item_ids.astype(jnp.int32)
    p = _pack_p(content_matrix.T, item_table.T, W.T, b)
    u_rows = _sc_user_gather(user_ids, user_table.T,
                             user_table.T[:, TAIL_LO:])
    pg = _sc_p_gather(item_ids, p)
    return _score(u_rows, pg)


# speedup vs baseline: 4.3993x; 1.4580x over previous
import functools

import jax
import jax.numpy as jnp
from jax import lax
from jax.experimental import pallas as pl
from jax.experimental.pallas import tpu as pltpu
from jax.experimental.pallas import tpu_sc as plsc

B = 16384
ED = 32
CD = 64
NU = 1000000
NI = 100000
PW = 128

OWN = 31232
STREAM = 32768
CHUNK = 2048
NCHUNK = STREAM // CHUNK
NG = 3
MAIN_END = 999936
TAIL_LO = NU - 128
NIDV = B // 16
CAP = 2048
CCAP = 256
DUMP = B


def _pack_body(cmT_ref, itT_ref, wT_ref, b_ref, p_ref):
    cm_blk = cmT_ref[...]
    proj = jax.lax.dot_general(
        cm_blk, wT_ref[...],
        dimension_numbers=(((0,), (1,)), ((), ())),
        preferred_element_type=jnp.float32,
    )
    proj = jnp.maximum(proj + b_ref[...][None, :], 0.0)
    item = itT_ref[...].T
    blk = item.shape[0]
    pad = jnp.zeros((blk, PW - 2 * ED), jnp.float32)
    p_ref[...] = jnp.concatenate([item, proj, pad], axis=1)


def _pack_p(cmT, itT, wT, b):
    blk = 12800
    grid = (NI + blk - 1) // blk
    return pl.pallas_call(
        _pack_body,
        grid=(grid,),
        in_specs=[
            pl.BlockSpec((CD, blk), lambda i: (0, i)),
            pl.BlockSpec((ED, blk), lambda i: (0, i)),
            pl.BlockSpec((ED, CD), lambda i: (0, 0)),
            pl.BlockSpec((ED,), lambda i: (0,)),
        ],
        out_specs=pl.BlockSpec((blk, PW), lambda i: (i, 0)),
        out_shape=jax.ShapeDtypeStruct((NI, PW), jnp.float32),
    )(cmT, itT, wT, b)


_info = plsc.get_sparse_core_info()
_NW = _info.num_cores * _info.num_subcores
_BPW = B // _NW


def _compact(src_ref, n_vregs, predicate, out_refs, make_vals, cap,
             unroll=None):
    def body(j, carry):
        c = src_ref[pl.ds(j * 16, 16)]
        m = predicate(c)
        cs = plsc.cumsum(jnp.where(m, 1, 0).astype(jnp.int32))
        pos = carry + cs - 1
        mm = m & (pos < cap)
        for out_ref, vals in zip(out_refs, make_vals(j, c)):
            plsc.store_scatter(out_ref, [pos], vals, mask=mm)
        return carry + lax.reduce_max(cs, (0,))

    if unroll is not None:
        return lax.fori_loop(0, n_vregs, body, jnp.int32(0), unroll=unroll)
    return lax.fori_loop(0, n_vregs, body, jnp.int32(0))


def _make_sc_user_gather():
    mesh = plsc.VectorSubcoreMesh(core_axis_name="c", subcore_axis_name="s")
    iota16 = lambda: lax.iota(jnp.int32, 16)

    @functools.partial(
        pl.kernel,
        mesh=mesh,
        compiler_params=pltpu.CompilerParams(
            use_tc_tiling_on_sc=True, needs_layout_passes=False),
        out_type=jax.ShapeDtypeStruct((B + 16, PW), jnp.float32),
        scratch_types=[
            pltpu.VMEM((B,), jnp.int32),
            pltpu.VMEM((CAP + 16,), jnp.int32),
            pltpu.VMEM((CAP + 16,), jnp.int32),
            pltpu.VMEM((CCAP + 16,), jnp.int32),
            pltpu.VMEM((CCAP + 16,), jnp.int32),
            pltpu.VMEM((ED, CHUNK), jnp.float32),
            pltpu.VMEM((ED, 128), jnp.float32),
            pltpu.VMEM((16, PW), jnp.float32),
            pltpu.VMEM((16, PW), jnp.float32),
            pltpu.VMEM((8, 16), jnp.int32),
            pltpu.VMEM((8, 16), jnp.int32),
            pltpu.SemaphoreType.DMA,
            pltpu.SemaphoreType.DMA,
            pltpu.SemaphoreType.DMA,
        ],
    )
    def user_gather(ids, tableT, tailT, out, ids_v, cbuf, bbuf, ccb, bcb,
                    slab_a, tail_v, rows_0, rows_1, idxr_0, idxr_1,
                    s_a, s_b, s_out):
        wid = lax.axis_index("s") * _info.num_cores + lax.axis_index("c")
        lo = wid * OWN
        hi = jnp.where(wid == _NW - 1, NU, lo + OWN)
        off = jnp.minimum(lo, MAIN_END - STREAM)

        def issue(slab, sem, c_lo):
            return pltpu.async_copy(
                tableT.at[:, pl.ds(pl.multiple_of(c_lo, 128), CHUNK)],
                slab, sem)

        pltpu.sync_copy(ids, ids_v)
        pltpu.sync_copy(tailT, tail_v)

        n = _compact(
            ids_v, NIDV, lambda c: (c >= lo) & (c < hi),
            [cbuf, bbuf],
            lambda j, c: [c, j * 16 + iota16()],
            CAP,
        )
        n = jnp.minimum(n, CAP)
        nv = (n + 15) // 16
        sent = n + lax.iota(jnp.int32, 16)
        plsc.store_scatter(cbuf, [sent], jnp.full((16,), -1, jnp.int32))
        plsc.store_scatter(bbuf, [sent], jnp.full((16,), DUMP, jnp.int32))

        rows_p = [rows_0, rows_1]
        idxr_p = [idxr_0, idxr_1]

        def process(slab, c_lo, width, par, drain):
            rows_v, idxr = rows_p[par], idxr_p[par]

            if drain is True:
                for g in range(NG):
                    pltpu.make_async_copy(
                        rows_v.at[pl.ds(g * 16, 16)],
                        out.at[idxr.at[g]], s_out).wait()
            elif drain is not False:
                @pl.when(drain)
                def _():
                    for g in range(NG):
                        pltpu.make_async_copy(
                            rows_v.at[pl.ds(g * 16, 16)],
                            out.at[idxr.at[g]], s_out).wait()

            m = _compact(
                cbuf, nv,
                lambda c: (c >= c_lo) & (c < c_lo + width),
                [ccb, bcb],
                lambda j, c: [c - c_lo,
                              plsc.load_gather(bbuf, [j * 16 + iota16()])],
                CCAP,
            )
            m = jnp.minimum(m, NG * 16)

            def group(g, _):
                lmask = g * 16 + iota16() < m
                cc = plsc.load_gather(ccb, [g * 16 + iota16()])
                bb = plsc.load_gather(bcb, [g * 16 + iota16()])
                cc = jnp.where(lmask, cc, 0)
                bb = jnp.where(lmask, bb, DUMP)
                for f in range(ED):
                    vals = plsc.load_gather(
                        slab, [jnp.full((16,), f, jnp.int32), cc])
                    plsc.store_scatter(
                        rows_v, [iota16(), jnp.full((16,), f, jnp.int32)],
                        vals)
                idxr[0, :] = bb
                pltpu.async_copy(
                    rows_v, out.at[idxr.at[0]], s_out).wait()
                return 0

            lax.fori_loop(0, (m + 15) // 16, group, 0)

        def chunk_body(ti, _):
            c_lo = off + ti * CHUNK
            pltpu.sync_copy(
                tableT.at[:, pl.ds(pl.multiple_of(c_lo, 128), CHUNK)],
                slab_a)
            process(slab_a, c_lo, CHUNK, 0, False)
            return 0

        lax.fori_loop(0, NCHUNK, chunk_body, 0)

        process(tail_v, TAIL_LO, 128, NCHUNK % 2, False)


    return user_gather


def _make_sc_p_gather():
    mesh = plsc.VectorSubcoreMesh(core_axis_name="c", subcore_axis_name="s")

    @functools.partial(
        pl.kernel,
        mesh=mesh,
        compiler_params=pltpu.CompilerParams(use_tc_tiling_on_sc=True),
        out_type=jax.ShapeDtypeStruct((B, PW), jnp.float32),
        scratch_types=[
            pltpu.VMEM((_BPW,), jnp.int32),
            pltpu.VMEM((_BPW, PW), jnp.float32),
            pltpu.SemaphoreType.DMA,
        ],
    )
    def p_gather(ids, ptable, out, idx_v, rows_v, sem):
        wid = lax.axis_index("s") * _info.num_cores + lax.axis_index("c")
        base = wid * _BPW
        pltpu.sync_copy(ids.at[pl.ds(base, _BPW)], idx_v)
        pltpu.async_copy(ptable.at[idx_v], rows_v, sem).wait()
        pltpu.sync_copy(rows_v, out.at[pl.ds(base, _BPW)])

    return p_gather


_sc_user_gather = _make_sc_user_gather()
_sc_p_gather = _make_sc_p_gather()


def _score_body(u_ref, pg_ref, out_ref):
    u = u_ref[:, :ED]
    s = pg_ref[:, :ED] + pg_ref[:, ED:2 * ED]
    out_ref[...] = jnp.sum(u * s, axis=1)


def _score(u_rows, pg):
    blk = 2048
    grid = B // blk
    return pl.pallas_call(
        _score_body,
        grid=(grid,),
        in_specs=[
            pl.BlockSpec((blk, PW), lambda i: (i, 0)),
            pl.BlockSpec((blk, PW), lambda i: (i, 0)),
        ],
        out_specs=pl.BlockSpec((blk,), lambda i: (i,)),
        out_shape=jax.ShapeDtypeStruct((B,), jnp.float32),
    )(u_rows, pg)


def kernel(user_ids, item_ids, user_table, item_table, content_matrix, W, b):
    user_ids = user_ids.astype(jnp.int32)
    item_ids = item_ids.astype(jnp.int32)
    p = _pack_p(content_matrix.T, item_table.T, W.T, b)
    u_rows = _sc_user_gather(user_ids, user_table.T,
                             user_table.T[:, TAIL_LO:])
    pg = _sc_p_gather(item_ids, p)
    return _score(u_rows, pg)
